# baseline (speedup 1.0000x reference)
import functools

import jax
import jax.numpy as jnp
from jax import lax
from jax.experimental import pallas as pl
from jax.experimental.pallas import tpu as pltpu
from jax.experimental.pallas import tpu_sc as plsc

_LANES = 16
_NUM_CORES = 2
_NUM_SUBCORES = 16
_NUM_WORKERS = _NUM_CORES * _NUM_SUBCORES


@functools.lru_cache(maxsize=None)
def _build_gather(batch: int, nnodes: int):
  chunk = batch // _NUM_WORKERS
  mesh = plsc.VectorSubcoreMesh(core_axis_name="c", subcore_axis_name="s")

  @functools.partial(
      pl.kernel,
      mesh=mesh,
      out_type=jax.ShapeDtypeStruct((batch,), jnp.float32),
      compiler_params=pltpu.CompilerParams(needs_layout_passes=False),
      scratch_types=[
          pltpu.VMEM((chunk,), jnp.int32),
          pltpu.VMEM((chunk,), jnp.float32),
          pltpu.VMEM((nnodes,), jnp.float32),
          pltpu.VMEM((_LANES,), jnp.float32),
          pltpu.SemaphoreType.DMA,
          pltpu.SemaphoreType.DMA,
          pltpu.SemaphoreType.DMA,
      ],
  )
  def gather_kernel(idx_hbm, w_hbm, b_hbm, out_hbm, idx_v, vals_v, w_v, b_v,
                    sem_i, sem_w, sem_b):
    wid = lax.axis_index("s") * _NUM_CORES + lax.axis_index("c")
    base = wid * chunk
    cp_idx = pltpu.async_copy(idx_hbm.at[pl.ds(base, chunk)], idx_v, sem_i)
    cp_w = pltpu.async_copy(w_hbm, w_v, sem_w)
    cp_b = pltpu.async_copy(b_hbm, b_v, sem_b)
    cp_idx.wait()
    cp_w.wait()
    cp_b.wait()
    bias = b_v[...]
    for j in range(chunk // _LANES):
      sl = pl.ds(j * _LANES, _LANES)
      vals_v[sl] = plsc.load_gather(w_v, [idx_v[sl]]) + bias
    pltpu.sync_copy(vals_v, out_hbm.at[pl.ds(base, chunk)])

  return gather_kernel


def kernel(n, embedding, W, b):
  del embedding
  batch = n.shape[0]
  nnodes = W.shape[1]
  idx = n.astype(jnp.int32)
  w_flat = W.reshape(nnodes).astype(jnp.float32)
  b_vec = jnp.broadcast_to(b.astype(jnp.float32), (_LANES,))
  out = _build_gather(batch, nnodes)(idx, w_flat, b_vec)
  return out.reshape(batch, 1)


# speedup vs baseline: 1.0679x; 1.0679x over previous
import functools

import jax
import jax.numpy as jnp
from jax import lax
from jax.experimental import pallas as pl
from jax.experimental.pallas import tpu as pltpu
from jax.experimental.pallas import tpu_sc as plsc

_LANES = 16
_NUM_CORES = 2
_NUM_SUBCORES = 16
_NUM_WORKERS = _NUM_CORES * _NUM_SUBCORES


@functools.lru_cache(maxsize=None)
def _build_gather(batch: int):
  chunk = batch // _NUM_WORKERS
  mesh = plsc.VectorSubcoreMesh(core_axis_name="c", subcore_axis_name="s")

  @functools.partial(
      pl.kernel,
      mesh=mesh,
      out_type=jax.ShapeDtypeStruct((batch,), jnp.float32),
      scratch_types=[
          pltpu.VMEM((chunk,), jnp.int32),
          pltpu.VMEM((chunk,), jnp.float32),
          pltpu.VMEM((_LANES,), jnp.float32),
          pltpu.SemaphoreType.DMA,
          pltpu.SemaphoreType.DMA,
          pltpu.SemaphoreType.DMA,
      ],
  )
  def gather_kernel(idx_hbm, w_hbm, b_hbm, out_hbm, idx_v, vals_v, b_v,
                    sem_i, sem_b, sem_g):
    wid = lax.axis_index("s") * _NUM_CORES + lax.axis_index("c")
    base = wid * chunk
    cp_idx = pltpu.async_copy(idx_hbm.at[pl.ds(base, chunk)], idx_v, sem_i)
    cp_b = pltpu.async_copy(b_hbm, b_v, sem_b)
    cp_idx.wait()
    cp_g = pltpu.async_copy(w_hbm.at[idx_v], vals_v, sem_g)
    cp_b.wait()
    cp_g.wait()
    bias = b_v[...]
    for j in range(chunk // _LANES):
      sl = pl.ds(j * _LANES, _LANES)
      vals_v[sl] = vals_v[sl] + bias
    pltpu.sync_copy(vals_v, out_hbm.at[pl.ds(base, chunk)])

  return gather_kernel


def kernel(n, embedding, W, b):
  del embedding
  batch = n.shape[0]
  nnodes = W.shape[1]
  idx = n.astype(jnp.int32)
  w_flat = W.reshape(nnodes).astype(jnp.float32)
  b_vec = jnp.broadcast_to(b.astype(jnp.float32), (_LANES,))
  out = _build_gather(batch)(idx, w_flat, b_vec)
  return out.reshape(batch, 1)
